# Initial kernel scaffold; baseline (speedup 1.0000x reference)
import jax
import jax.numpy as jnp
from jax.experimental import pallas as pl


def kernel(indices, input_embeds):
    raise NotImplementedError("write your pallas kernel here")


# speedup vs baseline: 4.6607x; 4.6607x over previous
import functools

import jax
import jax.numpy as jnp
from jax import lax
from jax.experimental import pallas as pl
from jax.experimental.pallas import tpu as pltpu
from jax.experimental.pallas import tpu_sc as plsc

VOCAB = 100000
DIM = 64
ROWS = 4096 * 50
IDX_MINOR = 128
NUM_WORKERS = 32
ROWS_PER_W = ROWS // NUM_WORKERS
IDX_ROWS_PER_W = ROWS_PER_W // IDX_MINOR
GATHERS_PER_GROUP = 10
GROUP_ROWS = GATHERS_PER_GROUP * IDX_MINOR
NUM_GROUPS = ROWS_PER_W // GROUP_ROWS


def _make_kernel():
    mesh = plsc.VectorSubcoreMesh(core_axis_name="c", subcore_axis_name="s")

    @functools.partial(
        pl.kernel,
        mesh=mesh,
        out_type=jax.ShapeDtypeStruct((ROWS, DIM), jnp.float32),
        scratch_types=[
            pltpu.VMEM((IDX_ROWS_PER_W, IDX_MINOR), jnp.int32),
            pltpu.VMEM((GROUP_ROWS, DIM), jnp.float32),
            pltpu.SemaphoreType.DMA,
        ],
        compiler_params=pltpu.CompilerParams(use_tc_tiling_on_sc=False),
    )
    def k(idx_hbm, table_hbm, out_hbm, idx_v, rows_v, sem):
        wid = lax.axis_index("s") * 2 + lax.axis_index("c")
        out_base = wid * ROWS_PER_W

        pltpu.sync_copy(idx_hbm.at[wid], idx_v)

        def body(g, carry):
            descs = []
            for j in range(GATHERS_PER_GROUP):
                descs.append(
                    pltpu.async_copy(
                        table_hbm.at[idx_v.at[g * GATHERS_PER_GROUP + j]],
                        rows_v.at[pl.ds(j * IDX_MINOR, IDX_MINOR)],
                        sem,
                    )
                )
            for d in descs:
                d.wait()
            pltpu.sync_copy(
                rows_v,
                out_hbm.at[pl.ds(out_base + g * GROUP_ROWS, GROUP_ROWS)],
            )
            return carry

        lax.fori_loop(0, NUM_GROUPS, body, 0)

    return k


_gather_kernel = _make_kernel()


def kernel(indices, input_embeds):
    idx3d = indices.astype(jnp.int32).reshape(
        NUM_WORKERS, IDX_ROWS_PER_W, IDX_MINOR
    )
    out = _gather_kernel(idx3d, input_embeds)
    return out.reshape(indices.shape[0], indices.shape[1], DIM)
